# speedup vs baseline: 4.4023x; 4.4023x over previous
import functools

import jax
import jax.numpy as jnp
from jax import lax
from jax.experimental import pallas as pl
from jax.experimental.pallas import tpu as pltpu
from jax.experimental.pallas import tpu_sc as plsc

N = 10000
E = 320000
D = 128
B = 100
NODES = 100

NC = 2
NS = 16
CH = 128
NCH = 79
E_PAD = NC * NS * NCH * CH
N_PAD = 10240
RPT = N_PAD // NS

_mesh = plsc.VectorSubcoreMesh(core_axis_name="c", subcore_axis_name="s")


@functools.partial(
    pl.kernel,
    out_type=[
        jax.ShapeDtypeStruct((NC * N_PAD, D), jnp.float32),
        jax.ShapeDtypeStruct((NC * N_PAD,), jnp.float32),
    ],
    mesh=_mesh,
    scratch_types=[
        pltpu.VMEM_SHARED((N_PAD, D), jnp.float32),
        pltpu.VMEM_SHARED((N_PAD,), jnp.float32),
        pltpu.VMEM((CH,), jnp.int32),
        pltpu.VMEM((CH,), jnp.int32),
        pltpu.VMEM((CH, D), jnp.float32),
        pltpu.VMEM((CH,), jnp.float32),
        pltpu.SemaphoreType.DMA,
    ],
)
def _sc_conv_agg(x_hbm, src_hbm, dst_hbm, zrow_hbm, zdeg_hbm,
                 agg_hbm, deg_hbm,
                 acc, dacc, src_v, dst_v, rows_v, ones_v, sem):
    c = lax.axis_index("c")
    s = lax.axis_index("s")
    wid = c * NS + s

    pltpu.sync_copy(zrow_hbm.at[pl.ds(s * RPT, RPT)], acc.at[pl.ds(s * RPT, RPT)])
    pltpu.sync_copy(zdeg_hbm.at[pl.ds(s * RPT, RPT)], dacc.at[pl.ds(s * RPT, RPT)])
    for i in range(CH // 16):
        ones_v[pl.ds(i * 16, 16)] = jnp.ones((16,), jnp.float32)
    plsc.subcore_barrier()

    def step(k, carry):
        base = wid * (NCH * CH) + k * CH
        pltpu.sync_copy(src_hbm.at[pl.ds(base, CH)], src_v)
        pltpu.sync_copy(dst_hbm.at[pl.ds(base, CH)], dst_v)
        pltpu.async_copy(x_hbm.at[src_v], rows_v, sem).wait()
        pltpu.sync_copy(rows_v, acc.at[dst_v], add=True)
        pltpu.sync_copy(ones_v, dacc.at[dst_v], add=True)
        return carry

    lax.fori_loop(0, NCH, step, 0)
    plsc.subcore_barrier()

    out0 = c * N_PAD + s * RPT
    pltpu.sync_copy(acc.at[pl.ds(s * RPT, RPT)], agg_hbm.at[pl.ds(out0, RPT)])
    pltpu.sync_copy(dacc.at[pl.ds(s * RPT, RPT)], deg_hbm.at[pl.ds(out0, RPT)])


def _tc_conv_body(agg_ref, deg_ref, W_ref, b_ref, g_ref, be_ref, out_ref):
    a = agg_ref[0, :N, :] + agg_ref[1, :N, :]
    dg = deg_ref[0, :N, :] + deg_ref[1, :N, :]
    a = a * (1.0 / jnp.maximum(dg, 1.0))
    h = jnp.dot(a, W_ref[...], preferred_element_type=jnp.float32) + b_ref[...]
    m = jnp.mean(h, axis=0, keepdims=True)
    v = jnp.mean(h * h, axis=0, keepdims=True) - m * m
    hn = g_ref[...] * (h - m) * lax.rsqrt(v + 1e-5) + be_ref[...]
    out_ref[...] = jnp.maximum(hn, 0.0)


_tc_conv = pl.pallas_call(
    _tc_conv_body,
    out_shape=jax.ShapeDtypeStruct((N, D), jnp.float32),
    in_specs=[pl.BlockSpec(memory_space=pltpu.VMEM)] * 6,
    out_specs=pl.BlockSpec(memory_space=pltpu.VMEM),
)


def _tc_final_body(agg_ref, deg_ref, batch_ref,
                   W_ref, b_ref, g_ref, be_ref,
                   Ws1_ref, bs1_ref, Ws2_ref, bs2_ref,
                   Wh1_ref, bh1_ref, Wh2_ref, bh2_ref, Wh3_ref, bh3_ref,
                   out_ref):
    a = agg_ref[0, :N, :] + agg_ref[1, :N, :]
    dg = deg_ref[0, :N, :] + deg_ref[1, :N, :]
    a = a * (1.0 / jnp.maximum(dg, 1.0))
    h = jnp.dot(a, W_ref[...], preferred_element_type=jnp.float32) + b_ref[...]
    m = jnp.mean(h, axis=0, keepdims=True)
    v = jnp.mean(h * h, axis=0, keepdims=True) - m * m
    hn = g_ref[...] * (h - m) * lax.rsqrt(v + 1e-5) + be_ref[...]
    h2 = jnp.maximum(hn, 0.0)

    gids = lax.broadcasted_iota(jnp.int32, (B, N), 0)
    pmat = jnp.where(gids == batch_ref[...], 1.0 / NODES, 0.0)
    xg = jnp.dot(pmat, h2, preferred_element_type=jnp.float32)

    sh = jnp.maximum(xg, 0.0)
    sh = jnp.dot(sh, Ws1_ref[...], preferred_element_type=jnp.float32) + bs1_ref[...]
    sh = jnp.maximum(jnp.dot(sh, Ws2_ref[...], preferred_element_type=jnp.float32) + bs2_ref[...], 0.0)
    o = jnp.maximum(jnp.dot(sh, Wh1_ref[...], preferred_element_type=jnp.float32) + bh1_ref[...], 0.0)
    o = jnp.maximum(jnp.dot(o, Wh2_ref[...], preferred_element_type=jnp.float32) + bh2_ref[...], 0.0)
    out_ref[...] = jnp.dot(o, Wh3_ref[...], preferred_element_type=jnp.float32) + bh3_ref[...]


_tc_final = pl.pallas_call(
    _tc_final_body,
    out_shape=jax.ShapeDtypeStruct((B, 10), jnp.float32),
    in_specs=[pl.BlockSpec(memory_space=pltpu.VMEM)] * 17,
    out_specs=pl.BlockSpec(memory_space=pltpu.VMEM),
)


def kernel(x, edge_index, batch, W1, b1, g1, be1, W2, b2, g2, be2,
           Ws1, bs1, Ws2, bs2, Wh1, bh1, Wh2, bh2, Wh3, bh3):
    pad = E_PAD - E
    srcp = jnp.concatenate([edge_index[0], jnp.zeros((pad,), jnp.int32)])
    dstp = jnp.concatenate([edge_index[1], jnp.full((pad,), N, jnp.int32)])
    zrow = jnp.zeros((N_PAD, D), jnp.float32)
    zdeg = jnp.zeros((N_PAD,), jnp.float32)

    agg1, deg1 = _sc_conv_agg(x, srcp, dstp, zrow, zdeg)
    agg1 = agg1.reshape(NC, N_PAD, D)
    deg1 = deg1.reshape(NC, N_PAD, 1)
    h1 = _tc_conv(agg1, deg1, W1, b1[None, :], g1[None, :], be1[None, :])

    agg2, deg2 = _sc_conv_agg(h1, srcp, dstp, zrow, zdeg)
    agg2 = agg2.reshape(NC, N_PAD, D)
    deg2 = deg2.reshape(NC, N_PAD, 1)
    return _tc_final(agg2, deg2, batch[None, :],
                     W2, b2[None, :], g2[None, :], be2[None, :],
                     Ws1, bs1[None, :], Ws2, bs2[None, :],
                     Wh1, bh1[None, :], Wh2, bh2[None, :],
                     Wh3, bh3[None, :])


# speedup vs baseline: 4.9279x; 1.1194x over previous
import functools

import jax
import jax.numpy as jnp
from jax import lax
from jax.experimental import pallas as pl
from jax.experimental.pallas import tpu as pltpu
from jax.experimental.pallas import tpu_sc as plsc

N = 10000
E = 320000
D = 128
B = 100
NODES = 100

NC = 2
NS = 16
CH = 96
NCH = 106
E_PAD = NC * NS * NCH * CH
N_PAD = 10240
RPT = N_PAD // NS

_mesh = plsc.VectorSubcoreMesh(core_axis_name="c", subcore_axis_name="s")


def _sc_body(with_deg, x_hbm, src_hbm, dst_hbm, zrow_hbm, zdeg_hbm,
             agg_hbm, deg_hbm,
             acc, dacc, srcs, dsts, rows0, rows1, dst_cur,
             ones_v, sem0, sem1):
    c = lax.axis_index("c")
    s = lax.axis_index("s")
    wid = c * NS + s

    pltpu.sync_copy(src_hbm.at[pl.ds(wid * NCH * CH, NCH * CH)], srcs)
    pltpu.sync_copy(dst_hbm.at[pl.ds(wid * NCH * CH, NCH * CH)], dsts)

    pltpu.sync_copy(zrow_hbm.at[pl.ds(s * RPT, RPT)], acc.at[pl.ds(s * RPT, RPT)])
    pltpu.sync_copy(zdeg_hbm.at[pl.ds(s * RPT, RPT)], dacc.at[pl.ds(s * RPT, RPT)])
    for i in range(CH // 16):
        ones_v[pl.ds(i * 16, 16)] = jnp.ones((16,), jnp.float32)
    plsc.subcore_barrier()

    bufs = ((rows0, sem0), (rows1, sem1))

    def gather(g, rv, sm):
        return pltpu.make_async_copy(x_hbm.at[srcs.at[pl.ds(g * CH, CH)]], rv, sm)

    gather(0, rows0, sem0).start()
    gather(1, rows1, sem1).start()

    def step(k2, carry):
        for b, (rv, sm) in enumerate(bufs):
            g = k2 * 2 + b
            gather(g, rv, sm).wait()
            for i in range(CH // 16):
                dst_cur[pl.ds(i * 16, 16)] = dsts[pl.ds(g * CH + i * 16, 16)]
            pltpu.sync_copy(rv, acc.at[dst_cur], add=True)
            if with_deg:
                pltpu.sync_copy(ones_v, dacc.at[dst_cur], add=True)

            @pl.when(g + 2 < NCH)
            def _():
                gather(g + 2, rv, sm).start()
        return carry

    lax.fori_loop(0, NCH // 2, step, 0)
    plsc.subcore_barrier()

    out0 = c * N_PAD + s * RPT
    pltpu.sync_copy(acc.at[pl.ds(s * RPT, RPT)], agg_hbm.at[pl.ds(out0, RPT)])
    if with_deg:
        pltpu.sync_copy(dacc.at[pl.ds(s * RPT, RPT)], deg_hbm.at[pl.ds(out0, RPT)])


def _make_sc(with_deg):
    return functools.partial(
        pl.kernel,
        out_type=[
            jax.ShapeDtypeStruct((NC * N_PAD, D), jnp.float32),
            jax.ShapeDtypeStruct((NC * N_PAD,), jnp.float32),
        ],
        mesh=_mesh,
        scratch_types=[
            pltpu.VMEM_SHARED((N_PAD, D), jnp.float32),
            pltpu.VMEM_SHARED((N_PAD,), jnp.float32),
            pltpu.VMEM((NCH * CH,), jnp.int32),
            pltpu.VMEM((NCH * CH,), jnp.int32),
            pltpu.VMEM((CH, D), jnp.float32),
            pltpu.VMEM((CH, D), jnp.float32),
            pltpu.VMEM((CH,), jnp.int32),
            pltpu.VMEM((CH,), jnp.float32),
            pltpu.SemaphoreType.DMA,
            pltpu.SemaphoreType.DMA,
        ],
    )(functools.partial(_sc_body, with_deg))


_sc_conv1 = _make_sc(True)
_sc_conv2 = _make_sc(False)


def _tc_conv_body(agg_ref, deg_ref, W_ref, b_ref, g_ref, be_ref, out_ref):
    a = agg_ref[0, :N, :] + agg_ref[1, :N, :]
    dg = deg_ref[0, :N, :] + deg_ref[1, :N, :]
    a = a * (1.0 / jnp.maximum(dg, 1.0))
    h = jnp.dot(a, W_ref[...], preferred_element_type=jnp.float32) + b_ref[...]
    m = jnp.mean(h, axis=0, keepdims=True)
    v = jnp.mean(h * h, axis=0, keepdims=True) - m * m
    hn = g_ref[...] * (h - m) * lax.rsqrt(v + 1e-5) + be_ref[...]
    out_ref[...] = jnp.maximum(hn, 0.0)


_tc_conv = pl.pallas_call(
    _tc_conv_body,
    out_shape=jax.ShapeDtypeStruct((N, D), jnp.float32),
    in_specs=[pl.BlockSpec(memory_space=pltpu.VMEM)] * 6,
    out_specs=pl.BlockSpec(memory_space=pltpu.VMEM),
)


def _tc_final_body(agg_ref, deg_ref, batch_ref,
                   W_ref, b_ref, g_ref, be_ref,
                   Ws1_ref, bs1_ref, Ws2_ref, bs2_ref,
                   Wh1_ref, bh1_ref, Wh2_ref, bh2_ref, Wh3_ref, bh3_ref,
                   out_ref):
    a = agg_ref[0, :N, :] + agg_ref[1, :N, :]
    dg = deg_ref[0, :N, :] + deg_ref[1, :N, :]
    a = a * (1.0 / jnp.maximum(dg, 1.0))
    h = jnp.dot(a, W_ref[...], preferred_element_type=jnp.float32) + b_ref[...]
    m = jnp.mean(h, axis=0, keepdims=True)
    v = jnp.mean(h * h, axis=0, keepdims=True) - m * m
    hn = g_ref[...] * (h - m) * lax.rsqrt(v + 1e-5) + be_ref[...]
    h2 = jnp.maximum(hn, 0.0)

    gids = lax.broadcasted_iota(jnp.int32, (B, N), 0)
    pmat = jnp.where(gids == batch_ref[...], 1.0 / NODES, 0.0)
    xg = jnp.dot(pmat, h2, preferred_element_type=jnp.float32)

    sh = jnp.maximum(xg, 0.0)
    sh = jnp.dot(sh, Ws1_ref[...], preferred_element_type=jnp.float32) + bs1_ref[...]
    sh = jnp.maximum(jnp.dot(sh, Ws2_ref[...], preferred_element_type=jnp.float32) + bs2_ref[...], 0.0)
    o = jnp.maximum(jnp.dot(sh, Wh1_ref[...], preferred_element_type=jnp.float32) + bh1_ref[...], 0.0)
    o = jnp.maximum(jnp.dot(o, Wh2_ref[...], preferred_element_type=jnp.float32) + bh2_ref[...], 0.0)
    out_ref[...] = jnp.dot(o, Wh3_ref[...], preferred_element_type=jnp.float32) + bh3_ref[...]


_tc_final = pl.pallas_call(
    _tc_final_body,
    out_shape=jax.ShapeDtypeStruct((B, 10), jnp.float32),
    in_specs=[pl.BlockSpec(memory_space=pltpu.VMEM)] * 17,
    out_specs=pl.BlockSpec(memory_space=pltpu.VMEM),
)


def kernel(x, edge_index, batch, W1, b1, g1, be1, W2, b2, g2, be2,
           Ws1, bs1, Ws2, bs2, Wh1, bh1, Wh2, bh2, Wh3, bh3):
    pad = E_PAD - E
    srcp = jnp.concatenate([edge_index[0], jnp.zeros((pad,), jnp.int32)])
    trash = N + (jnp.arange(pad, dtype=jnp.int32) % (N_PAD - N))
    dstp = jnp.concatenate([edge_index[1], trash])

    zrow = jnp.zeros((N_PAD, D), jnp.float32)
    zdeg = jnp.zeros((N_PAD,), jnp.float32)

    agg1, deg1 = _sc_conv1(x, srcp, dstp, zrow, zdeg)
    agg1 = agg1.reshape(NC, N_PAD, D)
    deg1 = deg1.reshape(NC, N_PAD, 1)
    h1 = _tc_conv(agg1, deg1, W1, b1[None, :], g1[None, :], be1[None, :])

    agg2, _ = _sc_conv2(h1, srcp, dstp, zrow, zdeg)
    agg2 = agg2.reshape(NC, N_PAD, D)
    return _tc_final(agg2, deg1, batch[None, :],
                     W2, b2[None, :], g2[None, :], be2[None, :],
                     Ws1, bs1[None, :], Ws2, bs2[None, :],
                     Wh1, bh1[None, :], Wh2, bh2[None, :],
                     Wh3, bh3[None, :])
